# baseline (speedup 1.0000x reference)
import functools

import jax
import jax.numpy as jnp
from jax import lax
from jax.experimental import pallas as pl
from jax.experimental.pallas import tpu as pltpu
from jax.experimental.pallas import tpu_sc as plsc

NC = 2
NS = 16
NW = NC * NS
LANES = 16

BATCH = 16384
DIM = 128
B_PER_W = BATCH // NW
CHUNK = 256
N_CHUNKS = B_PER_W // CHUNK


def _sc_body(x_hbm, idx_hbm, table_hbm, icept_hbm, out_hbm,
             idx_v, x_v, rows_v, bias_v, out_v, sem_t, sem_b):
    wid = lax.axis_index("s") * NC + lax.axis_index("c")
    lane_iota = lax.iota(jnp.int32, LANES)

    def chunk_step(c, _):
        base = wid * B_PER_W + c * CHUNK

        pltpu.sync_copy(idx_hbm.at[pl.ds(base, CHUNK)], idx_v)
        row_gather = pltpu.async_copy(table_hbm.at[idx_v], rows_v, sem_t)
        bias_gather = pltpu.async_copy(icept_hbm.at[idx_v], bias_v, sem_b)
        pltpu.sync_copy(x_hbm.at[pl.ds(base, CHUNK)], x_v)
        row_gather.wait()
        bias_gather.wait()

        def group_step(g, _):
            gbase = g * LANES
            rows16 = gbase + lane_iota
            acc0 = bias_v[pl.ds(gbase, LANES)]

            def d_step(d, acc):
                dcol = jnp.full((LANES,), d, jnp.int32)
                xcol = plsc.load_gather(x_v, [rows16, dcol])
                tcol = plsc.load_gather(rows_v, [rows16, dcol])
                return acc + xcol * tcol

            acc = lax.fori_loop(0, DIM, d_step, acc0, unroll=8)
            out_v[pl.ds(gbase, LANES)] = acc
            return 0

        lax.fori_loop(0, CHUNK // LANES, group_step, 0)
        pltpu.sync_copy(out_v, out_hbm.at[pl.ds(base, CHUNK)])
        return 0

    lax.fori_loop(0, N_CHUNKS, chunk_step, 0)


@jax.jit
def _alo_forward(x, index, table, icept):
    mesh = plsc.VectorSubcoreMesh(
        core_axis_name="c", subcore_axis_name="s",
        num_cores=NC, num_subcores=NS)
    run = pl.kernel(
        _sc_body,
        out_type=jax.ShapeDtypeStruct((BATCH,), jnp.float32),
        mesh=mesh,
        compiler_params=pltpu.CompilerParams(needs_layout_passes=False),
        scratch_types=[
            pltpu.VMEM((CHUNK,), jnp.int32),
            pltpu.VMEM((CHUNK, DIM), jnp.float32),
            pltpu.VMEM((CHUNK, DIM), jnp.float32),
            pltpu.VMEM((CHUNK,), jnp.float32),
            pltpu.VMEM((CHUNK,), jnp.float32),
            pltpu.SemaphoreType.DMA,
            pltpu.SemaphoreType.DMA,
        ],
    )
    return run(x, index, table, icept)


def kernel(x, index, treeplus_loo_layer, treeplus_loo_intercept):
    index = index.astype(jnp.int32)
    return _alo_forward(x, index, treeplus_loo_layer, treeplus_loo_intercept)


# speedup vs baseline: 2.3102x; 2.3102x over previous
import functools

import jax
import jax.numpy as jnp
from jax import lax
from jax.experimental import pallas as pl
from jax.experimental.pallas import tpu as pltpu
from jax.experimental.pallas import tpu_sc as plsc

NC = 2
NS = 16
NW = NC * NS
LANES = 16

BATCH = 16384
DIM = 128
B_PER_W = BATCH // NW
CHUNK = 256
N_CHUNKS = B_PER_W // CHUNK


def _sc_body(x_hbm, idx_hbm, table_hbm, icept_hbm, out_hbm,
             idx_v, x_v, rows_v, bias_v, out_v, tr_v, sem_t, sem_b):
    wid = lax.axis_index("s") * NC + lax.axis_index("c")
    lane_iota = lax.iota(jnp.int32, LANES)

    def chunk_step(c, _):
        base = wid * B_PER_W + c * CHUNK

        pltpu.sync_copy(idx_hbm.at[pl.ds(base, CHUNK)], idx_v)
        row_gather = pltpu.async_copy(table_hbm.at[idx_v], rows_v, sem_t)
        bias_gather = pltpu.async_copy(icept_hbm.at[idx_v], bias_v, sem_b)
        pltpu.sync_copy(x_hbm.at[pl.ds(base, CHUNK)], x_v)
        row_gather.wait()
        bias_gather.wait()

        def group_step(g, _):
            gbase = g * LANES
            for j in range(LANES):
                r = gbase + j
                acc = x_v[r, pl.ds(0, LANES)] * rows_v[r, pl.ds(0, LANES)]
                for di in range(1, DIM // LANES):
                    acc = acc + (x_v[r, pl.ds(di * LANES, LANES)] *
                                 rows_v[r, pl.ds(di * LANES, LANES)])
                tr_v[j, pl.ds(0, LANES)] = acc
            outacc = bias_v[pl.ds(gbase, LANES)]
            for c in range(LANES):
                col = jnp.full((LANES,), c, jnp.int32)
                outacc = outacc + plsc.load_gather(tr_v, [lane_iota, col])
            out_v[pl.ds(gbase, LANES)] = outacc
            return 0

        lax.fori_loop(0, CHUNK // LANES, group_step, 0)
        pltpu.sync_copy(out_v, out_hbm.at[pl.ds(base, CHUNK)])
        return 0

    lax.fori_loop(0, N_CHUNKS, chunk_step, 0)


@jax.jit
def _alo_forward(x, index, table, icept):
    mesh = plsc.VectorSubcoreMesh(
        core_axis_name="c", subcore_axis_name="s",
        num_cores=NC, num_subcores=NS)
    run = pl.kernel(
        _sc_body,
        out_type=jax.ShapeDtypeStruct((BATCH,), jnp.float32),
        mesh=mesh,
        compiler_params=pltpu.CompilerParams(needs_layout_passes=False),
        scratch_types=[
            pltpu.VMEM((CHUNK,), jnp.int32),
            pltpu.VMEM((CHUNK, DIM), jnp.float32),
            pltpu.VMEM((CHUNK, DIM), jnp.float32),
            pltpu.VMEM((CHUNK,), jnp.float32),
            pltpu.VMEM((CHUNK,), jnp.float32),
            pltpu.VMEM((LANES, LANES + 1), jnp.float32),
            pltpu.SemaphoreType.DMA,
            pltpu.SemaphoreType.DMA,
        ],
    )
    return run(x, index, table, icept)


def kernel(x, index, treeplus_loo_layer, treeplus_loo_intercept):
    index = index.astype(jnp.int32)
    return _alo_forward(x, index, treeplus_loo_layer, treeplus_loo_intercept)
